# baseline (speedup 1.0000x reference)
import functools

import jax
import jax.numpy as jnp
from jax import lax
from jax.experimental import pallas as pl
from jax.experimental.pallas import tpu as pltpu
from jax.experimental.pallas import tpu_sc as plsc

_VOCAB = 1000000
_EMB = 16
_B = 16384

_NC = 2
_NS = 16
_NW = _NC * _NS
_BPW = _B // _NW
_CHUNK = 128
_NCHUNK = _BPW // _CHUNK
_ROWS_PER_BLOCK = 128 // _EMB
_OUT_ROWS_W = _BPW * _EMB // 128

_mesh = plsc.VectorSubcoreMesh(core_axis_name="c", subcore_axis_name="s")


@functools.partial(
    pl.kernel,
    mesh=_mesh,
    out_type=jax.ShapeDtypeStruct((_B * _EMB // 128, 128), jnp.float32),
    scratch_types=[
        pltpu.VMEM((_NCHUNK, _CHUNK), jnp.int32),
        pltpu.VMEM((_NCHUNK, _CHUNK), jnp.int32),
        pltpu.VMEM((_BPW, 128), jnp.float32),
        pltpu.VMEM((_OUT_ROWS_W, 128), jnp.float32),
        pltpu.SemaphoreType.DMA,
    ],
    compiler_params=pltpu.CompilerParams(needs_layout_passes=False),
)
def _sc_gather(idx_hbm, table_hbm, out_hbm, idx_v, blk_v, rows_v, out_v, sem):
    wid = lax.axis_index("s") * _NC + lax.axis_index("c")
    pltpu.sync_copy(idx_hbm.at[pl.ds(wid * _NCHUNK, _NCHUNK)], idx_v)
    for i in range(_NCHUNK):
        for j in range(_CHUNK // 16):
            sl = pl.ds(j * 16, 16)
            blk_v[i, sl] = lax.shift_right_logical(idx_v[i, sl], 3)
    copies = []
    for i in range(_NCHUNK):
        copies.append(
            pltpu.async_copy(
                table_hbm.at[blk_v.at[i]],
                rows_v.at[pl.ds(i * _CHUNK, _CHUNK)],
                sem,
            )
        )
    lane = lax.iota(jnp.int32, 16)
    for i in range(_NCHUNK):
        copies[i].wait()
        for t in range(i * _CHUNK // 16, (i + 1) * _CHUNK // 16):
            idx16 = idx_v[t // 8, pl.ds((t % 8) * 16, 16)]
            cbase = lax.shift_left(lax.bitwise_and(idx16, 7), 4)
            rvec = lane + (t * 16)
            dbase = lax.shift_left(rvec, 4)
            for l in range(_EMB):
                v = plsc.load_gather(rows_v, [rvec, cbase + l])
                dflat = dbase + l
                plsc.store_scatter(
                    out_v,
                    [lax.shift_right_logical(dflat, 7),
                     lax.bitwise_and(dflat, 127)],
                    v,
                )
    pltpu.sync_copy(out_v, out_hbm.at[pl.ds(wid * _OUT_ROWS_W, _OUT_ROWS_W)])


def kernel(x_sparse, x_dense, table_item):
    idx = x_sparse[:, 0].reshape(_B // 128, 128)
    table2 = table_item.reshape(_VOCAB * _EMB // 128, 128)
    out = _sc_gather(idx, table2)
    return out.reshape(_B, _EMB)


# speedup vs baseline: 4.7459x; 4.7459x over previous
import functools

import jax
import jax.numpy as jnp
from jax import lax
from jax.experimental import pallas as pl
from jax.experimental.pallas import tpu as pltpu
from jax.experimental.pallas import tpu_sc as plsc

_VOCAB = 1000000
_EMB = 16
_B = 16384

_NC = 2
_NS = 16
_NW = _NC * _NS
_BPW = _B // _NW
_WAVE = 16
_NWAVE = _BPW // _WAVE

_mesh = plsc.VectorSubcoreMesh(core_axis_name="c", subcore_axis_name="s")


@functools.partial(
    pl.kernel,
    mesh=_mesh,
    out_type=jax.ShapeDtypeStruct((_EMB, _B), jnp.float32),
    scratch_types=[
        pltpu.VMEM((_BPW // 128, 128), jnp.int32),
        pltpu.VMEM((_WAVE * _EMB, 128), jnp.float32),
        pltpu.VMEM((_EMB, 64), jnp.float32),
        pltpu.VMEM((_EMB, _BPW), jnp.float32),
        pltpu.SemaphoreType.DMA,
    ],
    compiler_params=pltpu.CompilerParams(needs_layout_passes=False),
)
def _sc_gather(idx_hbm, table_t_hbm, out_hbm, idx_v, ring_v, tail_v, out_v, sem):
    wid = lax.axis_index("s") * _NC + lax.axis_index("c")
    pltpu.sync_copy(idx_hbm.at[pl.ds(wid * (_BPW // 128), _BPW // 128)], idx_v)
    _TAIL = (_VOCAB // 128) * 128
    _LASTBLK = _TAIL - 128
    pltpu.sync_copy(table_t_hbm.at[:, pl.ds(_TAIL, _VOCAB - _TAIL)], tail_v)
    lane = lax.iota(jnp.int32, 16)

    def wave(w, carry):
        vec = idx_v[w // 8, pl.ds((w % 8) * 16, 16)]
        copies = []
        for k in range(_WAVE):
            v = vec[k]
            vblk = lax.min(
                lax.shift_left(lax.shift_right_logical(v, 7), 7),
                jnp.int32(_LASTBLK),
            )
            copies.append(
                pltpu.async_copy(
                    table_t_hbm.at[:, pl.ds(pl.multiple_of(vblk, 128), 128)],
                    ring_v.at[pl.ds(k * _EMB, _EMB), :],
                    sem,
                )
            )
        for c in copies:
            c.wait()
        col16 = lax.bitwise_and(vec, 127)
        tcol16 = lax.clamp(jnp.int32(0), vec - _TAIL, jnp.int32(63))
        istail = jnp.where(vec >= _TAIL, jnp.int32(1), jnp.int32(0))
        for k in range(_WAVE):
            cols = lax.broadcast(col16[k], (16,))
            gathered = plsc.load_gather(ring_v, [lane + (k * _EMB), cols])
            tgathered = plsc.load_gather(tail_v, [lane, lax.broadcast(tcol16[k], (16,))])
            picked = lax.select(
                lax.broadcast(istail[k], (16,)) != 0, tgathered, gathered
            )
            outcols = lax.broadcast(w * _WAVE + k, (16,))
            plsc.store_scatter(out_v, [lane, outcols], picked)
        return carry

    lax.fori_loop(0, _NWAVE, wave, 0)
    pltpu.sync_copy(out_v, out_hbm.at[:, pl.ds(wid * _BPW, _BPW)])


def kernel(x_sparse, x_dense, table_item):
    idx = x_sparse[:, 0].reshape(_B // 128, 128)
    out = _sc_gather(idx, table_item.T)
    return out.T


# speedup vs baseline: 6.8350x; 1.4402x over previous
import functools

import jax
import jax.numpy as jnp
from jax import lax
from jax.experimental import pallas as pl
from jax.experimental.pallas import tpu as pltpu
from jax.experimental.pallas import tpu_sc as plsc

_VOCAB = 1000000
_EMB = 16
_B = 16384

_NC = 2
_NS = 16
_NW = _NC * _NS
_BPW = _B // _NW
_WAVE = 16
_NWAVE = _BPW // _WAVE
_TAIL = (_VOCAB // 128) * 128
_LASTBLK = _TAIL - 128

_mesh = plsc.VectorSubcoreMesh(core_axis_name="c", subcore_axis_name="s")


@functools.partial(
    pl.kernel,
    mesh=_mesh,
    out_type=jax.ShapeDtypeStruct((_EMB, _B), jnp.float32),
    scratch_types=[
        pltpu.VMEM((_BPW // 128, 128), jnp.int32),
        pltpu.VMEM((2 * _WAVE * _EMB, 128), jnp.float32),
        pltpu.VMEM((_EMB, 64), jnp.float32),
        pltpu.VMEM((_EMB, _BPW), jnp.float32),
        pltpu.SemaphoreType.DMA,
    ],
    compiler_params=pltpu.CompilerParams(needs_layout_passes=False),
)
def _sc_gather(idx_hbm, table_t_hbm, out_hbm, idx_v, ring_v, tail_v, out_v, sem):
    wid = lax.axis_index("s") * _NC + lax.axis_index("c")
    pltpu.sync_copy(idx_hbm.at[pl.ds(wid * (_BPW // 128), _BPW // 128)], idx_v)
    pltpu.sync_copy(table_t_hbm.at[:, pl.ds(_TAIL, _VOCAB - _TAIL)], tail_v)
    lane = lax.iota(jnp.int32, 16)
    half_words = _WAVE * _EMB

    def load_vec(w):
        return idx_v[w // 8, pl.ds((w % 8) * 16, 16)]

    def fire(w, vec):
        base = (w % 2) * half_words
        for k in range(_WAVE):
            v = vec[k]
            vblk = lax.min(
                lax.shift_left(lax.shift_right_logical(v, 7), 7),
                jnp.int32(_LASTBLK),
            )
            pltpu.async_copy(
                table_t_hbm.at[:, pl.ds(pl.multiple_of(vblk, 128), 128)],
                ring_v.at[pl.ds(base + k * _EMB, _EMB), :],
                sem,
            )

    def extract(w, vec):
        base = (w % 2) * half_words
        for k in range(_WAVE):
            pltpu.make_async_copy(
                table_t_hbm.at[:, pl.ds(0, 128)],
                ring_v.at[pl.ds(base + k * _EMB, _EMB), :],
                sem,
            ).wait()
        col16 = lax.bitwise_and(vec, 127)
        tcol16 = lax.clamp(jnp.int32(0), vec - _TAIL, jnp.int32(63))
        istail = jnp.where(vec >= _TAIL, jnp.int32(1), jnp.int32(0))
        for k in range(_WAVE):
            cols = lax.broadcast(col16[k], (16,))
            gathered = plsc.load_gather(ring_v, [lane + (base + k * _EMB), cols])
            tgathered = plsc.load_gather(
                tail_v, [lane, lax.broadcast(tcol16[k], (16,))]
            )
            picked = lax.select(
                lax.broadcast(istail[k], (16,)) != 0, tgathered, gathered
            )
            outcols = lax.broadcast(w * _WAVE + k, (16,))
            plsc.store_scatter(out_v, [lane, outcols], picked)

    vec0 = load_vec(0)
    fire(0, vec0)

    def wave(w, vec_prev):
        vec_w = load_vec(w)
        fire(w, vec_w)
        extract(w - 1, vec_prev)
        return vec_w

    vec_last = lax.fori_loop(1, _NWAVE, wave, vec0)
    extract(_NWAVE - 1, vec_last)
    pltpu.sync_copy(out_v, out_hbm.at[:, pl.ds(wid * _BPW, _BPW)])


def kernel(x_sparse, x_dense, table_item):
    idx = x_sparse[:, 0].reshape(_B // 128, 128)
    out = _sc_gather(idx, table_item.T)
    return out.T


# speedup vs baseline: 6.9130x; 1.0114x over previous
import functools

import jax
import jax.numpy as jnp
from jax import lax
from jax.experimental import pallas as pl
from jax.experimental.pallas import tpu as pltpu
from jax.experimental.pallas import tpu_sc as plsc

_VOCAB = 1000000
_EMB = 16
_B = 16384

_NC = 2
_NS = 16
_NW = _NC * _NS
_BPW = _B // _NW
_WAVE = 16
_NWAVE = _BPW // _WAVE
_TAIL = (_VOCAB // 128) * 128
_LASTBLK = _TAIL - 128

_mesh = plsc.VectorSubcoreMesh(core_axis_name="c", subcore_axis_name="s")


@functools.partial(
    pl.kernel,
    mesh=_mesh,
    out_type=jax.ShapeDtypeStruct((_EMB, _B), jnp.float32),
    scratch_types=[
        pltpu.VMEM((_BPW // 128, 128), jnp.int32),
        pltpu.VMEM((3 * _WAVE * _EMB, 128), jnp.float32),
        pltpu.VMEM((_EMB, 64), jnp.float32),
        pltpu.VMEM((_EMB, _BPW), jnp.float32),
        pltpu.SemaphoreType.DMA,
    ],
    compiler_params=pltpu.CompilerParams(needs_layout_passes=False),
)
def _sc_gather(idx_hbm, table_t_hbm, out_hbm, idx_v, ring_v, tail_v, out_v, sem):
    wid = lax.axis_index("s") * _NC + lax.axis_index("c")
    pltpu.sync_copy(idx_hbm.at[pl.ds(wid * (_BPW // 128), _BPW // 128)], idx_v)
    pltpu.sync_copy(table_t_hbm.at[:, pl.ds(_TAIL, _VOCAB - _TAIL)], tail_v)
    lane = lax.iota(jnp.int32, 16)
    half_words = _WAVE * _EMB

    def load_vec(w):
        return idx_v[w // 8, pl.ds((w % 8) * 16, 16)]

    def fire(w, vec):
        base = (w % 3) * half_words
        for k in range(_WAVE):
            v = vec[k]
            vblk = lax.min(
                lax.shift_left(lax.shift_right_logical(v, 7), 7),
                jnp.int32(_LASTBLK),
            )
            pltpu.async_copy(
                table_t_hbm.at[:, pl.ds(pl.multiple_of(vblk, 128), 128)],
                ring_v.at[pl.ds(base + k * _EMB, _EMB), :],
                sem,
            )

    def extract(w, vec):
        base = (w % 3) * half_words
        for k in range(_WAVE):
            pltpu.make_async_copy(
                table_t_hbm.at[:, pl.ds(0, 128)],
                ring_v.at[pl.ds(base + k * _EMB, _EMB), :],
                sem,
            ).wait()
        col16 = lax.bitwise_and(vec, 127)
        tcol16 = lax.clamp(jnp.int32(0), vec - _TAIL, jnp.int32(63))
        istail = jnp.where(vec >= _TAIL, jnp.int32(1), jnp.int32(0))
        for k in range(_WAVE):
            cols = lax.broadcast(col16[k], (16,))
            gathered = plsc.load_gather(ring_v, [lane + (base + k * _EMB), cols])
            tgathered = plsc.load_gather(
                tail_v, [lane, lax.broadcast(tcol16[k], (16,))]
            )
            picked = lax.select(
                lax.broadcast(istail[k], (16,)) != 0, tgathered, gathered
            )
            outcols = lax.broadcast(w * _WAVE + k, (16,))
            plsc.store_scatter(out_v, [lane, outcols], picked)

    vec0 = load_vec(0)
    fire(0, vec0)
    vec1 = load_vec(1)
    fire(1, vec1)

    def wave(w, carry):
        vec_pp, vec_p = carry
        vec_w = load_vec(w)
        fire(w, vec_w)
        extract(w - 2, vec_pp)
        return (vec_p, vec_w)

    vec_pp, vec_p = lax.fori_loop(2, _NWAVE, wave, (vec0, vec1))
    extract(_NWAVE - 2, vec_pp)
    extract(_NWAVE - 1, vec_p)
    pltpu.sync_copy(out_v, out_hbm.at[:, pl.ds(wid * _BPW, _BPW)])


def kernel(x_sparse, x_dense, table_item):
    idx = x_sparse[:, 0].reshape(_B // 128, 128)
    out = _sc_gather(idx, table_item.T)
    return out.T
